# baseline (speedup 1.0000x reference)
import jax
import jax.numpy as jnp
from jax.experimental import pallas as pl
from jax.experimental.pallas import tpu as pltpu

N = 5000
S = 16
H = 64
GD = 4 * H
B = 2560
GRID = 2
OVER = GRID * B - N


def _lstm_block_kernel(x_hbm, w0_ref, b0_ref, w1_ref, b1_ref, hw_ref,
                       out_ref, xbuf, sem):
    blk = pl.program_id(0)
    base = jnp.where(blk == 0, 0, N - B)

    def copy(t):
        return pltpu.make_async_copy(
            x_hbm.at[pl.ds(base, B), t],
            xbuf.at[t % 2],
            sem.at[t % 2])

    copy(0).start()

    w0 = w0_ref[...]
    b0 = b0_ref[...]
    w1 = w1_ref[...]
    b1 = b1_ref[...]

    def cell(w, b, xt, h, c):
        z = jnp.concatenate([xt, h], axis=1).astype(jnp.bfloat16)
        gates = b + jnp.dot(z, w, preferred_element_type=jnp.float32)
        sig_if = jax.nn.sigmoid(gates[:, 0:2 * H])
        t_go = jnp.tanh(gates[:, 2 * H:4 * H])
        i_ = sig_if[:, 0:H]
        f_ = sig_if[:, H:2 * H]
        g_ = t_go[:, 0:H]
        o_ = t_go[:, H:2 * H] * 0.5 + 0.5
        c = f_ * c + i_ * g_
        h = o_ * jnp.tanh(c)
        return h, c

    CH = 2
    Bc = B // CH
    z = jnp.zeros((Bc, H), jnp.float32)
    h0 = [z] * CH
    c0 = [z] * CH
    h1 = [z] * CH
    c1 = [z] * CH
    for t in range(S):
        if t + 1 < S:
            copy(t + 1).start()
        copy(t).wait()
        xt = xbuf[t % 2]
        for k in range(CH):
            xk = xt[k * Bc:(k + 1) * Bc]
            h0[k], c0[k] = cell(w0, b0, xk, h0[k], c0[k])
            h1[k], c1[k] = cell(w1, b1, h0[k], h1[k], c1[k])

    hf = jnp.concatenate(h1, axis=0)
    out_ref[...] = jnp.dot(hf, hw_ref[...],
                           preferred_element_type=jnp.float32)


def kernel(data, label, proposals, classes,
           w_ih_0, w_hh_0, b_ih_0, b_hh_0,
           w_ih_1, w_hh_1, b_ih_1, b_hh_1,
           cls_w, cls_b, bbox_w, bbox_b):
    f32 = jnp.float32

    def prep(w_ih, w_hh, b_ih, b_hh):
        wt = jnp.concatenate([w_ih, w_hh], axis=1).T
        scale = jnp.concatenate([jnp.ones((3 * H,), f32),
                                 jnp.full((H,), 0.5, f32)])
        wt = wt * scale[None, :]
        b = ((b_ih + b_hh) * scale).reshape(1, GD)
        return wt.astype(jnp.bfloat16), b

    w0, b0 = prep(w_ih_0, w_hh_0, b_ih_0, b_hh_0)
    w1, b1 = prep(w_ih_1, w_hh_1, b_ih_1, b_hh_1)
    hw = jnp.concatenate([cls_w, bbox_w, jnp.zeros((1, H), f32)], axis=0).T

    out = pl.pallas_call(
        _lstm_block_kernel,
        grid=(GRID,),
        in_specs=[
            pl.BlockSpec(memory_space=pltpu.MemorySpace.HBM),
            pl.BlockSpec((2 * H, GD), lambda i: (0, 0)),
            pl.BlockSpec((1, GD), lambda i: (0, 0)),
            pl.BlockSpec((2 * H, GD), lambda i: (0, 0)),
            pl.BlockSpec((1, GD), lambda i: (0, 0)),
            pl.BlockSpec((H, 8), lambda i: (0, 0)),
        ],
        out_specs=pl.BlockSpec((B, 8), lambda i: (i, 0)),
        out_shape=jax.ShapeDtypeStruct((GRID * B, 8), f32),
        scratch_shapes=[
            pltpu.VMEM((2, B, H), f32),
            pltpu.SemaphoreType.DMA((2,)),
        ],
        compiler_params=pltpu.CompilerParams(
            dimension_semantics=("parallel",)),
    )(proposals, w0, b0, w1, b1, hw)

    outN = jnp.concatenate([out[:B], out[B + OVER:]], axis=0)
    cls_feat = outN[:, :5] + cls_b
    bbox_feat = outN[:, 5:7] + bbox_b
    return (cls_feat, bbox_feat, jnp.float32(0.0), jnp.float32(0.0))


# speedup vs baseline: 1.0293x; 1.0293x over previous
import jax
import jax.numpy as jnp
from jax.experimental import pallas as pl
from jax.experimental.pallas import tpu as pltpu

N = 5000
S = 16
H = 64
GD = 4 * H
B = 2560
GRID = 2
OVER = GRID * B - N


def _lstm_block_kernel(x_hbm, w0_ref, b0_ref, w1_ref, b1_ref, hw_ref,
                       out_ref, xbuf, sem):
    blk = pl.program_id(0)
    base = jnp.where(blk == 0, 0, N - B)

    def copy(t):
        return pltpu.make_async_copy(
            x_hbm.at[pl.ds(base, B), t],
            xbuf.at[t % 2],
            sem.at[t % 2])

    copy(0).start()

    w0 = w0_ref[...]
    b0 = b0_ref[...]
    w1 = w1_ref[...]
    b1 = b1_ref[...]

    def cell(w, b, xt, h, c):
        z = jnp.concatenate([xt, h], axis=1).astype(jnp.bfloat16)
        gates = b + jnp.dot(z, w, preferred_element_type=jnp.float32)
        tg = jnp.tanh(gates)
        i_ = tg[:, 0:H] * 0.5 + 0.5
        f_ = tg[:, H:2 * H] * 0.5 + 0.5
        g_ = tg[:, 2 * H:3 * H]
        o_ = tg[:, 3 * H:4 * H] * 0.5 + 0.5
        c = f_ * c + i_ * g_
        h = o_ * jnp.tanh(c)
        return h, c

    CH = 10
    Bc = B // CH
    z = jnp.zeros((Bc, H), jnp.float32)
    h0 = [z] * CH
    c0 = [z] * CH
    h1 = [z] * CH
    c1 = [z] * CH
    for t in range(S):
        if t + 1 < S:
            copy(t + 1).start()
        copy(t).wait()
        xt = xbuf[t % 2]
        for k in range(CH):
            xk = xt[k * Bc:(k + 1) * Bc]
            h0[k], c0[k] = cell(w0, b0, xk, h0[k], c0[k])
            h1[k], c1[k] = cell(w1, b1, h0[k], h1[k], c1[k])

    hf = jnp.concatenate(h1, axis=0)
    out_ref[...] = jnp.dot(hf, hw_ref[...],
                           preferred_element_type=jnp.float32)


def kernel(data, label, proposals, classes,
           w_ih_0, w_hh_0, b_ih_0, b_hh_0,
           w_ih_1, w_hh_1, b_ih_1, b_hh_1,
           cls_w, cls_b, bbox_w, bbox_b):
    f32 = jnp.float32

    def prep(w_ih, w_hh, b_ih, b_hh):
        wt = jnp.concatenate([w_ih, w_hh], axis=1).T
        scale = jnp.concatenate([jnp.full((2 * H,), 0.5, f32),
                                 jnp.ones((H,), f32),
                                 jnp.full((H,), 0.5, f32)])
        wt = wt * scale[None, :]
        b = ((b_ih + b_hh) * scale).reshape(1, GD)
        return wt.astype(jnp.bfloat16), b

    w0, b0 = prep(w_ih_0, w_hh_0, b_ih_0, b_hh_0)
    w1, b1 = prep(w_ih_1, w_hh_1, b_ih_1, b_hh_1)
    hw = jnp.concatenate([cls_w, bbox_w, jnp.zeros((1, H), f32)], axis=0).T

    out = pl.pallas_call(
        _lstm_block_kernel,
        grid=(GRID,),
        in_specs=[
            pl.BlockSpec(memory_space=pltpu.MemorySpace.HBM),
            pl.BlockSpec((2 * H, GD), lambda i: (0, 0)),
            pl.BlockSpec((1, GD), lambda i: (0, 0)),
            pl.BlockSpec((2 * H, GD), lambda i: (0, 0)),
            pl.BlockSpec((1, GD), lambda i: (0, 0)),
            pl.BlockSpec((H, 8), lambda i: (0, 0)),
        ],
        out_specs=pl.BlockSpec((B, 8), lambda i: (i, 0)),
        out_shape=jax.ShapeDtypeStruct((GRID * B, 8), f32),
        scratch_shapes=[
            pltpu.VMEM((2, B, H), f32),
            pltpu.SemaphoreType.DMA((2,)),
        ],
        compiler_params=pltpu.CompilerParams(
            dimension_semantics=("parallel",)),
    )(proposals, w0, b0, w1, b1, hw)

    outN = jnp.concatenate([out[:B], out[B + OVER:]], axis=0)
    cls_feat = outN[:, :5] + cls_b
    bbox_feat = outN[:, 5:7] + bbox_b
    return (cls_feat, bbox_feat, jnp.float32(0.0), jnp.float32(0.0))


# speedup vs baseline: 1.0590x; 1.0289x over previous
import jax
import jax.numpy as jnp
from jax.experimental import pallas as pl
from jax.experimental.pallas import tpu as pltpu

N = 5000
S = 16
H = 64
GD = 4 * H
B = 2560
GRID = 2
OVER = GRID * B - N


def _lstm_block_kernel(x_hbm, w0_ref, w1_ref, hw_ref, hb_ref,
                       out_ref, xbuf, sem):
    blk = pl.program_id(0)
    base = jnp.where(blk == 0, 0, N - B)

    def copy(t):
        return pltpu.make_async_copy(
            x_hbm.at[pl.ds(base, B), t],
            xbuf.at[t % 2],
            sem.at[t % 2])

    copy(0).start()

    w0 = w0_ref[...]
    w1 = w1_ref[...]

    CH = 10
    Bc = B // CH
    ones = jnp.ones((Bc, 8), jnp.bfloat16)

    def cell(w, xt, h, c):
        z = jnp.concatenate([xt.astype(jnp.bfloat16),
                             h.astype(jnp.bfloat16), ones], axis=1)
        gates = jnp.dot(z, w, preferred_element_type=jnp.float32)
        tg = jnp.tanh(gates)
        i_ = tg[:, 0:H] * 0.5 + 0.5
        f_ = tg[:, H:2 * H] * 0.5 + 0.5
        g_ = tg[:, 2 * H:3 * H]
        o_ = tg[:, 3 * H:4 * H] * 0.5 + 0.5
        c = f_ * c + i_ * g_
        h = o_ * jnp.tanh(c)
        return h, c

    z = jnp.zeros((Bc, H), jnp.float32)
    h0 = [z] * CH
    c0 = [z] * CH
    h1 = [z] * CH
    c1 = [z] * CH
    for t in range(S):
        if t + 1 < S:
            copy(t + 1).start()
        copy(t).wait()
        xt = xbuf[t % 2]
        for k in range(CH):
            xk = xt[k * Bc:(k + 1) * Bc]
            h0[k], c0[k] = cell(w0, xk, h0[k], c0[k])
            h1[k], c1[k] = cell(w1, h0[k], h1[k], c1[k])

    hf = jnp.concatenate(h1, axis=0)
    out_ref[...] = jnp.dot(hf, hw_ref[...],
                           preferred_element_type=jnp.float32) + hb_ref[...]


def kernel(data, label, proposals, classes,
           w_ih_0, w_hh_0, b_ih_0, b_hh_0,
           w_ih_1, w_hh_1, b_ih_1, b_hh_1,
           cls_w, cls_b, bbox_w, bbox_b):
    f32 = jnp.float32

    def prep(w_ih, w_hh, b_ih, b_hh):
        scale = jnp.concatenate([jnp.full((2 * H,), 0.5, f32),
                                 jnp.ones((H,), f32),
                                 jnp.full((H,), 0.5, f32)])
        b = (b_ih + b_hh) * scale
        wt = jnp.concatenate(
            [jnp.concatenate([w_ih, w_hh], axis=1).T * scale[None, :],
             jnp.tile(b.reshape(1, GD) / 8.0, (8, 1))], axis=0)
        return wt.astype(jnp.bfloat16)

    w0 = prep(w_ih_0, w_hh_0, b_ih_0, b_hh_0)
    w1 = prep(w_ih_1, w_hh_1, b_ih_1, b_hh_1)
    hw = jnp.concatenate([cls_w, bbox_w, jnp.zeros((1, H), f32)], axis=0).T
    hb = jnp.concatenate([cls_b, bbox_b, jnp.zeros((1,), f32)]).reshape(1, 8)

    out = pl.pallas_call(
        _lstm_block_kernel,
        grid=(GRID,),
        in_specs=[
            pl.BlockSpec(memory_space=pltpu.MemorySpace.HBM),
            pl.BlockSpec((2 * H + 8, GD), lambda i: (0, 0)),
            pl.BlockSpec((2 * H + 8, GD), lambda i: (0, 0)),
            pl.BlockSpec((H, 8), lambda i: (0, 0)),
            pl.BlockSpec((1, 8), lambda i: (0, 0)),
        ],
        out_specs=pl.BlockSpec((B, 8), lambda i: (i, 0)),
        out_shape=jax.ShapeDtypeStruct((GRID * B, 8), f32),
        scratch_shapes=[
            pltpu.VMEM((2, B, H), f32),
            pltpu.SemaphoreType.DMA((2,)),
        ],
        compiler_params=pltpu.CompilerParams(
            dimension_semantics=("parallel",)),
    )(proposals, w0, w1, hw, hb)

    outN = jnp.concatenate([out[:B], out[B + OVER:]], axis=0)
    cls_feat = outN[:, :5]
    bbox_feat = outN[:, 5:7]
    return (cls_feat, bbox_feat, jnp.float32(0.0), jnp.float32(0.0))
